# Initial kernel scaffold; baseline (speedup 1.0000x reference)
import jax
import jax.numpy as jnp
from jax.experimental import pallas as pl


def kernel(x, edge_index, W1, a1s, a1d, W2, a2s, a2d, W3, a3s, a3d):
    raise NotImplementedError("write your pallas kernel here")


# speedup vs baseline: 14.6394x; 14.6394x over previous
import functools

import jax
import jax.numpy as jnp
from jax import lax
from jax.experimental import pallas as pl
from jax.experimental.pallas import tpu as pltpu
from jax.experimental.pallas import tpu_sc as plsc

NC = 2
NS = 16
LN = 16

NEG_SLOPE = 0.2
EPS = 1e-16
BLK = 512


def _elu(x):
    return jnp.where(x > 0, x, jnp.exp(jnp.minimum(x, 0.0)) - 1.0)


def _store_heads(hs, as_ref, ad_ref, hh_ref, es_ref, ed_ref):
    es, ed = [], []
    for h, hb in enumerate(hs):
        hh_ref[h] = hb
        es.append(jnp.dot(hb, as_ref[h][:, None], preferred_element_type=jnp.float32))
        ed.append(jnp.dot(hb, ad_ref[h][:, None], preferred_element_type=jnp.float32))
    es_ref[...] = jnp.concatenate(es, axis=1)
    ed_ref[...] = jnp.concatenate(ed, axis=1)


def _prep1_body(x_ref, w_ref, as_ref, ad_ref, hh_ref, es_ref, ed_ref):
    H = w_ref.shape[0]
    xb = x_ref[...]
    hs = [jnp.dot(xb, w_ref[h], preferred_element_type=jnp.float32)
          for h in range(H)]
    _store_heads(hs, as_ref, ad_ref, hh_ref, es_ref, ed_ref)


def _prep1(x, W, a_s, a_d):
    N, Fin = x.shape
    H, _, F = W.shape
    grid = (N // BLK,)
    return pl.pallas_call(
        _prep1_body,
        grid=grid,
        in_specs=[
            pl.BlockSpec((BLK, Fin), lambda i: (i, 0)),
            pl.BlockSpec((H, Fin, F), lambda i: (0, 0, 0)),
            pl.BlockSpec((H, F), lambda i: (0, 0)),
            pl.BlockSpec((H, F), lambda i: (0, 0)),
        ],
        out_specs=[
            pl.BlockSpec((H, BLK, F), lambda i: (0, i, 0)),
            pl.BlockSpec((BLK, H), lambda i: (i, 0)),
            pl.BlockSpec((BLK, H), lambda i: (i, 0)),
        ],
        out_shape=[
            jax.ShapeDtypeStruct((H, N, F), jnp.float32),
            jax.ShapeDtypeStruct((N, H), jnp.float32),
            jax.ShapeDtypeStruct((N, H), jnp.float32),
        ],
    )(x, W, a_s, a_d)


def _att_out(raw_ref, h, F):
    num = raw_ref[h, :, 0:F]
    den = raw_ref[h, :, F][:, None]
    return _elu(num / (den + EPS))


def _prep2_body(raw_ref, w_ref, as_ref, ad_ref, h1_ref, hh_ref, es_ref, ed_ref):
    H = w_ref.shape[0]
    F = (raw_ref.shape[-1] // 8) * 8 - 8
    x2 = jnp.concatenate([_att_out(raw_ref, h, F) for h in range(raw_ref.shape[0])],
                         axis=1)
    h1_ref[...] = x2
    hs = [jnp.dot(x2, w_ref[h], preferred_element_type=jnp.float32)
          for h in range(H)]
    _store_heads(hs, as_ref, ad_ref, hh_ref, es_ref, ed_ref)


def _prep2(raw1, W, a_s, a_d):
    Hin, N, FP = raw1.shape
    F = FP - 8
    H, Fin, Fo = W.shape
    grid = (N // BLK,)
    return pl.pallas_call(
        _prep2_body,
        grid=grid,
        in_specs=[
            pl.BlockSpec((Hin, BLK, FP), lambda i: (0, i, 0)),
            pl.BlockSpec((H, Fin, Fo), lambda i: (0, 0, 0)),
            pl.BlockSpec((H, Fo), lambda i: (0, 0)),
            pl.BlockSpec((H, Fo), lambda i: (0, 0)),
        ],
        out_specs=[
            pl.BlockSpec((BLK, Fin), lambda i: (i, 0)),
            pl.BlockSpec((H, BLK, Fo), lambda i: (0, i, 0)),
            pl.BlockSpec((BLK, H), lambda i: (i, 0)),
            pl.BlockSpec((BLK, H), lambda i: (i, 0)),
        ],
        out_shape=[
            jax.ShapeDtypeStruct((N, Fin), jnp.float32),
            jax.ShapeDtypeStruct((H, N, Fo), jnp.float32),
            jax.ShapeDtypeStruct((N, H), jnp.float32),
            jax.ShapeDtypeStruct((N, H), jnp.float32),
        ],
    )(raw1, W, a_s, a_d)


def _prep3_body(raw_ref, h1_ref, w_ref, as_ref, ad_ref, hh_ref, es_ref, ed_ref):
    H = w_ref.shape[0]
    F = (raw_ref.shape[-1] // 8) * 8 - 8
    x3 = jnp.concatenate([_att_out(raw_ref, h, F) for h in range(raw_ref.shape[0])],
                         axis=1)
    x3 = x3 + h1_ref[...]
    hs = [jnp.dot(x3, w_ref[h], preferred_element_type=jnp.float32)
          for h in range(H)]
    _store_heads(hs, as_ref, ad_ref, hh_ref, es_ref, ed_ref)


def _prep3(raw2, h1, W, a_s, a_d):
    Hin, N, FP = raw2.shape
    F = FP - 8
    H, Fin, Fo = W.shape
    grid = (N // BLK,)
    return pl.pallas_call(
        _prep3_body,
        grid=grid,
        in_specs=[
            pl.BlockSpec((Hin, BLK, FP), lambda i: (0, i, 0)),
            pl.BlockSpec((BLK, Fin), lambda i: (i, 0)),
            pl.BlockSpec((H, Fin, Fo), lambda i: (0, 0, 0)),
            pl.BlockSpec((H, Fo), lambda i: (0, 0)),
            pl.BlockSpec((H, Fo), lambda i: (0, 0)),
        ],
        out_specs=[
            pl.BlockSpec((H, BLK, Fo), lambda i: (0, i, 0)),
            pl.BlockSpec((BLK, H), lambda i: (i, 0)),
            pl.BlockSpec((BLK, H), lambda i: (i, 0)),
        ],
        out_shape=[
            jax.ShapeDtypeStruct((H, N, Fo), jnp.float32),
            jax.ShapeDtypeStruct((N, H), jnp.float32),
            jax.ShapeDtypeStruct((N, H), jnp.float32),
        ],
    )(raw2, h1, W, a_s, a_d)


def _final_body(raw_ref, out_ref):
    H = raw_ref.shape[0]
    F = (raw_ref.shape[-1] // 8) * 8 - 8
    acc = _att_out(raw_ref, 0, F)
    for h in range(1, H):
        acc = acc + _att_out(raw_ref, h, F)
    y = acc / float(H)
    y = y - jnp.max(y, axis=1, keepdims=True)
    e = jnp.exp(y)
    out_ref[...] = e / jnp.sum(e, axis=1, keepdims=True)


def _final(raw3):
    Hin, N, FP = raw3.shape
    F = FP - 8
    grid = (N // BLK,)
    return pl.pallas_call(
        _final_body,
        grid=grid,
        in_specs=[pl.BlockSpec((Hin, BLK, FP), lambda i: (0, i, 0))],
        out_specs=pl.BlockSpec((BLK, F), lambda i: (i, 0)),
        out_shape=jax.ShapeDtypeStruct((N, F), jnp.float32),
    )(raw3)


def _sc_attention(hh, es, ed, src3, dst3, e_real):
    H, N, F = hh.shape
    FP = F + 8
    Hp = H // NC
    B = 64 if F >= 128 else 128
    CHB = 2048 // B
    EPT = src3.shape[1] * src3.shape[2]
    NCH = EPT // 2048
    RPT = N // NS
    mesh = plsc.VectorSubcoreMesh(core_axis_name="c", subcore_axis_name="s")

    zoffs = list(range(0, FP - LN, LN)) + [FP - LN]

    @functools.partial(
        pl.kernel,
        out_type=jax.ShapeDtypeStruct((H, N, FP), jnp.float32),
        mesh=mesh,
        compiler_params=pltpu.CompilerParams(needs_layout_passes=False,
                                             use_tc_tiling_on_sc=False),
        scratch_types=[
            pltpu.VMEM((CHB, B), jnp.int32),
            pltpu.VMEM((CHB, B), jnp.int32),
            pltpu.VMEM((N,), jnp.float32),
            pltpu.VMEM((N,), jnp.float32),
            pltpu.VMEM((B,), jnp.float32),
            pltpu.VMEM((B, F), jnp.float32),
            pltpu.VMEM((B, FP), jnp.float32),
            pltpu.VMEM_SHARED((N, FP), jnp.float32),
            pltpu.SemaphoreType.DMA,
        ],
    )
    def sc_kernel(hh_ref, es_ref, ed_ref, src_ref, dst_ref, out_ref,
                  src_c, dst_c, es_v, ed_v, wbuf, gbuf, sbuf, acc, gsem):
        s = lax.axis_index("s")
        c = lax.axis_index("c")
        zv = jnp.zeros((LN,), jnp.float32)
        iota16 = lax.iota(jnp.int32, LN)
        colF = jnp.full((LN,), F, jnp.int32)
        base = s * RPT

        for k in range(Hp):
            h = c * Hp + k
            pltpu.sync_copy(es_ref.at[h], es_v)
            pltpu.sync_copy(ed_ref.at[h], ed_v)

            def zrow(r, _):
                for o in zoffs:
                    sbuf[r, o:o + LN] = zv
                return 0
            lax.fori_loop(0, B, zrow, 0)
            for i in range(RPT // B):
                pltpu.sync_copy(sbuf, acc.at[pl.ds(base + i * B, B)])
            plsc.subcore_barrier()

            def chunk(cc, _):
                pltpu.sync_copy(src_ref.at[s].at[pl.ds(cc * CHB, CHB)], src_c)
                pltpu.sync_copy(dst_ref.at[s].at[pl.ds(cc * CHB, CHB)], dst_c)

                def batch(jj, _):
                    gcp = pltpu.async_copy(hh_ref.at[h].at[src_c.at[jj]],
                                           gbuf, gsem)
                    for kk in range(B // LN):
                        sl = pl.ds(kk * LN, LN)
                        s16 = src_c[jj, sl]
                        d16 = dst_c[jj, sl]
                        ev = (plsc.load_gather(es_v, [s16])
                              + plsc.load_gather(ed_v, [d16]))
                        ev = jnp.where(ev < 0, ev * NEG_SLOPE, ev)
                        w = jnp.exp(ev)
                        gid = s * EPT + cc * 2048 + jj * B + kk * LN + iota16
                        w = jnp.where(gid < e_real, w, 0.0)
                        wbuf[sl] = w
                        plsc.store_scatter(sbuf, [iota16 + kk * LN, colF], w)
                    gcp.wait()

                    def edge(b, _):
                        wspl = plsc.load_gather(
                            wbuf, [jnp.full((LN,), b, jnp.int32)])
                        for k2 in range(F // LN):
                            sl2 = pl.ds(k2 * LN, LN)
                            sbuf[b, sl2] = gbuf[b, sl2] * wspl
                        return 0
                    lax.fori_loop(0, B, edge, 0)

                    pltpu.sync_copy(sbuf, acc.at[dst_c.at[jj]], add=True)
                    return 0
                lax.fori_loop(0, CHB, batch, 0)
                return 0
            lax.fori_loop(0, NCH, chunk, 0)

            plsc.subcore_barrier()
            pltpu.sync_copy(acc.at[pl.ds(base, RPT)],
                            out_ref.at[h].at[pl.ds(base, RPT)])
            plsc.subcore_barrier()

    return sc_kernel(hh, es, ed, src3, dst3)


def kernel(x, edge_index, W1, a1s, a1d, W2, a2s, a2d, W3, a3s, a3d):
    E = edge_index.shape[1]
    N = x.shape[0]
    NP = -(-N // 5120) * 5120
    src = edge_index[0]
    dst = edge_index[1]
    ept = -(-E // (NS * 2048)) * 2048
    pad = NS * ept - E
    srcp = jnp.pad(src, (0, pad))
    dstp = jnp.pad(dst, (0, pad))
    src64 = srcp.reshape(NS, ept // 64, 64)
    dst64 = dstp.reshape(NS, ept // 64, 64)
    src128 = srcp.reshape(NS, ept // 128, 128)
    dst128 = dstp.reshape(NS, ept // 128, 128)
    xp = jnp.pad(x, ((0, NP - N), (0, 0)))

    hh1, es1, ed1 = _prep1(xp, W1, a1s, a1d)
    raw1 = _sc_attention(hh1, es1.T, ed1.T, src64, dst64, E)
    h1, hh2, es2, ed2 = _prep2(raw1, W2, a2s, a2d)
    raw2 = _sc_attention(hh2, es2.T, ed2.T, src64, dst64, E)
    hh3, es3, ed3 = _prep3(raw2, h1, W3, a3s, a3d)
    raw3 = _sc_attention(hh3, es3.T, ed3.T, src128, dst128, E)
    return _final(raw3)[:N]
